# baseline (speedup 1.0000x reference)
import functools

import jax
import jax.numpy as jnp
from jax import lax
from jax.experimental import pallas as pl
from jax.experimental.pallas import tpu as pltpu
from jax.experimental.pallas import tpu_sc as plsc

N_NODES = 10000
NPAD = 10240
E_EDGES = 320000
D_IN = 128
H = 64
G = 64
GA = 4
BN_EPS = 1e-5

NUM_CORES = 2
NUM_SUBCORES = 16
NUM_TILES = NUM_CORES * NUM_SUBCORES
CH = 128
PER_TILE_CHUNKS = 80
PER_TILE_E = CH * PER_TILE_CHUNKS
EPAD = PER_TILE_E * NUM_TILES
ROWS_PER_TILE = NPAD // NUM_SUBCORES
RING = 8
GAH = 4
SDL = RING - GAH

BLK = 1024
NBLK = NPAD // BLK


@functools.cache
def _edge_agg_kernel():
    mesh = plsc.VectorSubcoreMesh(core_axis_name="c", subcore_axis_name="s")

    @functools.partial(
        pl.kernel,
        out_type=jax.ShapeDtypeStruct((NUM_CORES * NPAD, H), jnp.int16),
        mesh=mesh,
        scratch_types=[
            pltpu.VMEM((PER_TILE_CHUNKS, CH), jnp.int32),
            pltpu.VMEM((PER_TILE_CHUNKS, CH), jnp.int32),
            pltpu.VMEM((RING, CH, H), jnp.int16),
            pltpu.VMEM_SHARED((NPAD, H), jnp.int16),
            pltpu.VMEM_SHARED((NPAD, H), jnp.int16),
            pltpu.SemaphoreType.DMA((RING,)),
            pltpu.SemaphoreType.DMA((RING,)),
        ],
        compiler_params=pltpu.CompilerParams(use_tc_tiling_on_sc=False),
    )
    def body(y_hbm, src_hbm, dst_hbm, zeros_hbm, out_hbm,
             src_all, dst_all, rows, y_sh, acc_sh, gsem, ssem):
        cid = lax.axis_index("c")
        sid = lax.axis_index("s")
        tid = cid * NUM_SUBCORES + sid
        row0 = sid * ROWS_PER_TILE
        ch0 = tid * PER_TILE_CHUNKS
        pltpu.sync_copy(src_hbm.at[pl.ds(ch0, PER_TILE_CHUNKS)], src_all)
        pltpu.sync_copy(dst_hbm.at[pl.ds(ch0, PER_TILE_CHUNKS)], dst_all)
        pltpu.sync_copy(y_hbm.at[pl.ds(row0, ROWS_PER_TILE)],
                        y_sh.at[pl.ds(row0, ROWS_PER_TILE)])
        pltpu.sync_copy(zeros_hbm, acc_sh.at[pl.ds(row0, ROWS_PER_TILE)])

        def gsrc(b):
            return y_hbm if b in (3, 7) else y_sh

        def gfire(c, b):
            pltpu.async_copy(gsrc(b).at[src_all.at[c]], rows.at[b],
                             gsem.at[b])

        def gdrain(c, b):
            pltpu.make_async_copy(gsrc(b).at[src_all.at[c]],
                                  rows.at[b], gsem.at[b]).wait()

        def sfire(c, b):
            pltpu.async_copy(rows.at[b], acc_sh.at[dst_all.at[c]],
                             ssem.at[b], add=True)

        def sdrain(c, b):
            pltpu.make_async_copy(rows.at[b],
                                  acc_sh.at[dst_all.at[c]], ssem.at[b]).wait()

        plsc.subcore_barrier()
        for j in range(GAH):
            gfire(j, j)

        LAST_T = PER_TILE_CHUNKS // RING - 1

        def step(t, carry):
            c0 = t * RING
            for j in range(RING):
                c = c0 + j
                bg = (j + GAH) % RING
                if j >= SDL:
                    sdrain(c - SDL, bg)
                else:
                    @pl.when(t >= 1)
                    def _():
                        sdrain(c - SDL, bg)
                if LAST_T * RING + j + GAH < PER_TILE_CHUNKS:
                    gfire(c + GAH, bg)
                else:
                    @pl.when(t < LAST_T)
                    def _():
                        gfire(c + GAH, bg)
                gdrain(c, j)
                sfire(c, j)
            return carry

        lax.fori_loop(0, LAST_T + 1, step, 0)
        for j in range(SDL):
            c = PER_TILE_CHUNKS - SDL + j
            sdrain(c, c % RING)
        plsc.subcore_barrier()
        out0 = pl.multiple_of(cid * NPAD + row0, 8)
        pltpu.sync_copy(acc_sh.at[pl.ds(row0, ROWS_PER_TILE)],
                        out_hbm.at[pl.ds(out0, ROWS_PER_TILE)])

    return body


def _edge_agg(y, srcp, dstp, zeros_blk):
    return _edge_agg_kernel()(y, srcp, dstp, zeros_blk)


def _mm_body(x_ref, w_ref, o_ref, m_ref):
    i = pl.program_id(0)
    y = lax.dot(x_ref[...], w_ref[...],
                preferred_element_type=jnp.float32)
    o_ref[...] = y

    @pl.when(i == 0)
    def _():
        m_ref[...] = jnp.zeros_like(m_ref)

    m_ref[...] = jnp.maximum(m_ref[...], jnp.max(jnp.abs(y)))


def _matmul(x, w):
    n, d = x.shape
    h = w.shape[1]
    return pl.pallas_call(
        _mm_body,
        grid=(n // BLK,),
        in_specs=[pl.BlockSpec((BLK, d), lambda i: (i, 0)),
                  pl.BlockSpec((d, h), lambda i: (0, 0))],
        out_specs=[pl.BlockSpec((BLK, h), lambda i: (i, 0)),
                   pl.BlockSpec((1, 1), lambda i: (0, 0))],
        out_shape=[jax.ShapeDtypeStruct((n, h), jnp.float32),
                   jax.ShapeDtypeStruct((1, 1), jnp.float32)],
    )(x, w)


QSCALE = 1023.0


def _quant_body(y_ref, m_ref, q_ref):
    scale = QSCALE / jnp.maximum(m_ref[0, 0], 1e-30)
    q_ref[...] = jnp.round(y_ref[...] * scale).astype(jnp.int16)


def _quant(y, maxv):
    return pl.pallas_call(
        _quant_body,
        grid=(NBLK,),
        in_specs=[pl.BlockSpec((BLK, H), lambda i: (i, 0)),
                  pl.BlockSpec((1, 1), lambda i: (0, 0))],
        out_specs=pl.BlockSpec((BLK, H), lambda i: (i, 0)),
        out_shape=jax.ShapeDtypeStruct((NPAD, H), jnp.int16),
    )(y, maxv)


def _layer_body(y_ref, a0_ref, a1_ref, b_ref, c1_ref, s1_ref, d1_ref,
                w2_ref, d2_ref, w1n_ref, ynext_ref, pooled_ref, cnt_ref,
                m_ref):
    i = pl.program_id(0)
    y = y_ref[...]
    agg = a0_ref[...].astype(jnp.float32) + a1_ref[...].astype(jnp.float32)
    h1 = jnp.maximum(y * c1_ref[...] + agg * s1_ref[...] + d1_ref[...], 0.0)
    hh = jnp.maximum(
        lax.dot(h1, w2_ref[...],
                preferred_element_type=jnp.float32) + d2_ref[...], 0.0)
    ynext = lax.dot(hh, w1n_ref[...],
                    preferred_element_type=jnp.float32)
    ynext_ref[...] = ynext
    oh = jnp.equal(b_ref[...],
                   lax.broadcasted_iota(jnp.int32, (BLK, G), 1)
                   ).astype(jnp.float32)
    ph = lax.dot_general(oh, hh, (((0,), (0,)), ((), ())),
                         preferred_element_type=jnp.float32)
    pc = lax.dot_general(oh, jnp.ones((BLK, 1), jnp.float32),
                         (((0,), (0,)), ((), ())),
                         preferred_element_type=jnp.float32)

    @pl.when(i == 0)
    def _():
        pooled_ref[...] = jnp.zeros_like(pooled_ref)
        cnt_ref[...] = jnp.zeros_like(cnt_ref)
        m_ref[...] = jnp.zeros_like(m_ref)

    pooled_ref[...] += ph
    cnt_ref[...] += pc
    m_ref[...] = jnp.maximum(m_ref[...], jnp.max(jnp.abs(ynext)))


def _layer_call(y, aggs, batch2d, c1, s1, d1, w2p, d2, w1n):
    return pl.pallas_call(
        _layer_body,
        grid=(NBLK,),
        in_specs=[
            pl.BlockSpec((BLK, H), lambda i: (i, 0)),
            pl.BlockSpec((BLK, H), lambda i: (i, 0)),
            pl.BlockSpec((BLK, H), lambda i: (i + NBLK, 0)),
            pl.BlockSpec((BLK, 1), lambda i: (i, 0)),
            pl.BlockSpec((1, H), lambda i: (0, 0)),
            pl.BlockSpec((1, H), lambda i: (0, 0)),
            pl.BlockSpec((1, H), lambda i: (0, 0)),
            pl.BlockSpec((H, H), lambda i: (0, 0)),
            pl.BlockSpec((1, H), lambda i: (0, 0)),
            pl.BlockSpec((H, H), lambda i: (0, 0)),
        ],
        out_specs=[
            pl.BlockSpec((BLK, H), lambda i: (i, 0)),
            pl.BlockSpec((G, H), lambda i: (0, 0)),
            pl.BlockSpec((G, 1), lambda i: (0, 0)),
            pl.BlockSpec((1, 1), lambda i: (0, 0)),
        ],
        out_shape=[
            jax.ShapeDtypeStruct((NPAD, H), jnp.float32),
            jax.ShapeDtypeStruct((G, H), jnp.float32),
            jax.ShapeDtypeStruct((G, 1), jnp.float32),
            jax.ShapeDtypeStruct((1, 1), jnp.float32),
        ],
    )(y, aggs, aggs, batch2d, c1, s1, d1, w2p, d2, w1n)


def _clf_body(p0_ref, p1_ref, p2_ref, cnt_ref, g_ref,
              wa_ref, wb_ref, wc_ref, wg_ref, b1_ref, w2_ref, b2_ref, o_ref):
    inv = 1.0 / jnp.maximum(cnt_ref[...], 1.0)
    dot = functools.partial(lax.dot,
                            preferred_element_type=jnp.float32)
    e = (dot(p0_ref[...] * inv, wa_ref[...])
         + dot(p1_ref[...] * inv, wb_ref[...])
         + dot(p2_ref[...] * inv, wc_ref[...])
         + dot(g_ref[...], wg_ref[...])
         + b1_ref[...])
    hc = jnp.maximum(e, 0.0)
    o_ref[...] = dot(hc, w2_ref[...]) + b2_ref[...]


def _clf_call(p0, p1, p2, cnt, gp, wa, wb, wc, wg, b1, w2, b2):
    return pl.pallas_call(
        _clf_body,
        out_shape=jax.ShapeDtypeStruct((G, 128), jnp.float32),
    )(p0, p1, p2, cnt, gp, wa, wb, wc, wg, b1, w2, b2)


def kernel(x, edge_index, batch, graph_attr, params):
    layers = params['layers']
    clf = params['clf']

    xp = jnp.zeros((NPAD, D_IN), jnp.float32).at[:N_NODES].set(x)
    src = edge_index[0]
    dst = edge_index[1]
    pad_e = EPAD - E_EDGES
    srcp = jnp.concatenate(
        [src, jnp.zeros((pad_e,), jnp.int32)]).reshape(EPAD // CH, CH)
    dstp = jnp.concatenate(
        [dst, jnp.full((pad_e,), N_NODES, jnp.int32)]).reshape(EPAD // CH, CH)
    batchp = jnp.concatenate(
        [batch, jnp.full((NPAD - N_NODES,), G, jnp.int32)]).reshape(NPAD, 1)
    zeros_blk = jnp.zeros((ROWS_PER_TILE, H), jnp.int16)

    inv_bn = 1.0 / jnp.sqrt(1.0 + BN_EPS)
    pooled = []
    cnt = None
    y, ymax = _matmul(xp, layers[0]['W1'])
    for l in range(3):
        lp = layers[l]
        s1 = (lp['g1'] * inv_bn).reshape(1, H)
        c1 = (1.0 + lp['eps']) * s1
        s1q = s1 * (ymax[0, 0] / QSCALE)
        d1 = (lp['b1'] * lp['g1'] * inv_bn + lp['be1']).reshape(1, H)
        s2 = lp['go'] * inv_bn
        w2p = lp['W2'] * s2[None, :]
        d2 = (lp['b2'] * s2 + lp['bo']).reshape(1, H)
        w1n = layers[l + 1]['W1'] if l < 2 else jnp.zeros((H, H), jnp.float32)
        y16 = _quant(y, ymax)
        aggs = _edge_agg(y16, srcp, dstp, zeros_blk)
        y, psum, pc, ymax = _layer_call(y, aggs, batchp, c1, s1q, d1,
                                        w2p, d2, w1n)
        pooled.append(psum)
        if cnt is None:
            cnt = pc

    wc1 = clf['Wc1']
    wa, wb, wc = wc1[0:H], wc1[H:2 * H], wc1[2 * H:3 * H]
    wg = jnp.zeros((128, H), jnp.float32).at[:GA].set(wc1[3 * H:])
    gp = jnp.zeros((G, 128), jnp.float32).at[:, :GA].set(graph_attr)
    w2 = jnp.zeros((H, 128), jnp.float32).at[:, :2].set(clf['Wc2'])
    b2 = jnp.zeros((1, 128), jnp.float32).at[0, :2].set(clf['bc2'])
    out = _clf_call(pooled[0], pooled[1], pooled[2], cnt, gp,
                    wa, wb, wc, wg, clf['bc1'].reshape(1, H), w2, b2)
    return out[:, :2]


# speedup vs baseline: 1.1016x; 1.1016x over previous
import functools

import jax
import jax.numpy as jnp
from jax import lax
from jax.experimental import pallas as pl
from jax.experimental.pallas import tpu as pltpu
from jax.experimental.pallas import tpu_sc as plsc

N_NODES = 10000
NPAD = 10240
E_EDGES = 320000
D_IN = 128
H = 64
G = 64
GA = 4
BN_EPS = 1e-5

NUM_CORES = 2
NUM_SUBCORES = 16
NUM_TILES = NUM_CORES * NUM_SUBCORES
CH = 128
PER_TILE_CHUNKS = 80
PER_TILE_E = CH * PER_TILE_CHUNKS
EPAD = PER_TILE_E * NUM_TILES
ROWS_PER_TILE = NPAD // NUM_SUBCORES
RING = 8
GAH = 4
SDL = RING - GAH

BLK = 2048
NBLK = NPAD // BLK


@functools.cache
def _edge_agg_kernel():
    mesh = plsc.VectorSubcoreMesh(core_axis_name="c", subcore_axis_name="s")

    @functools.partial(
        pl.kernel,
        out_type=jax.ShapeDtypeStruct((NUM_CORES * NPAD, H), jnp.int16),
        mesh=mesh,
        scratch_types=[
            pltpu.VMEM((PER_TILE_CHUNKS, CH), jnp.int32),
            pltpu.VMEM((PER_TILE_CHUNKS, CH), jnp.int32),
            pltpu.VMEM((RING, CH, H), jnp.int16),
            pltpu.VMEM_SHARED((NPAD, H), jnp.int16),
            pltpu.VMEM_SHARED((NPAD, H), jnp.int16),
            pltpu.SemaphoreType.DMA((RING,)),
            pltpu.SemaphoreType.DMA((RING,)),
        ],
        compiler_params=pltpu.CompilerParams(use_tc_tiling_on_sc=False),
    )
    def body(y_hbm, src_hbm, dst_hbm, zeros_hbm, out_hbm,
             src_all, dst_all, rows, y_sh, acc_sh, gsem, ssem):
        cid = lax.axis_index("c")
        sid = lax.axis_index("s")
        tid = cid * NUM_SUBCORES + sid
        row0 = sid * ROWS_PER_TILE
        ch0 = tid * PER_TILE_CHUNKS
        pltpu.sync_copy(src_hbm.at[pl.ds(ch0, PER_TILE_CHUNKS)], src_all)
        pltpu.sync_copy(dst_hbm.at[pl.ds(ch0, PER_TILE_CHUNKS)], dst_all)
        pltpu.sync_copy(y_hbm.at[pl.ds(row0, ROWS_PER_TILE)],
                        y_sh.at[pl.ds(row0, ROWS_PER_TILE)])
        pltpu.sync_copy(zeros_hbm, acc_sh.at[pl.ds(row0, ROWS_PER_TILE)])

        def gfire(c, b):
            pltpu.async_copy(y_sh.at[src_all.at[c]], rows.at[b], gsem.at[b])

        def gdrain(c, b):
            pltpu.make_async_copy(y_sh.at[src_all.at[c]],
                                  rows.at[b], gsem.at[b]).wait()

        def sfire(c, b):
            pltpu.async_copy(rows.at[b], acc_sh.at[dst_all.at[c]],
                             ssem.at[b], add=True)

        def sdrain(c, b):
            pltpu.make_async_copy(rows.at[b],
                                  acc_sh.at[dst_all.at[c]], ssem.at[b]).wait()

        plsc.subcore_barrier()
        for j in range(GAH):
            gfire(j, j)

        LAST_T = PER_TILE_CHUNKS // RING - 1

        def step(t, carry):
            c0 = t * RING
            for j in range(RING):
                c = c0 + j
                bg = (j + GAH) % RING
                if j >= SDL:
                    sdrain(c - SDL, bg)
                else:
                    @pl.when(t >= 1)
                    def _():
                        sdrain(c - SDL, bg)
                if LAST_T * RING + j + GAH < PER_TILE_CHUNKS:
                    gfire(c + GAH, bg)
                else:
                    @pl.when(t < LAST_T)
                    def _():
                        gfire(c + GAH, bg)
                gdrain(c, j)
                sfire(c, j)
            return carry

        lax.fori_loop(0, LAST_T + 1, step, 0)
        for j in range(SDL):
            c = PER_TILE_CHUNKS - SDL + j
            sdrain(c, c % RING)
        plsc.subcore_barrier()
        out0 = pl.multiple_of(cid * NPAD + row0, 8)
        pltpu.sync_copy(acc_sh.at[pl.ds(row0, ROWS_PER_TILE)],
                        out_hbm.at[pl.ds(out0, ROWS_PER_TILE)])

    return body


def _edge_agg(y, srcp, dstp, zeros_blk):
    return _edge_agg_kernel()(y, srcp, dstp, zeros_blk)


def _mm_body(x_ref, w_ref, o_ref, m_ref):
    i = pl.program_id(0)
    y = lax.dot(x_ref[...], w_ref[...],
                preferred_element_type=jnp.float32)
    o_ref[...] = y

    @pl.when(i == 0)
    def _():
        m_ref[...] = jnp.zeros_like(m_ref)

    m_ref[...] = jnp.maximum(m_ref[...], jnp.max(jnp.abs(y)))


def _matmul(x, w):
    n, d = x.shape
    h = w.shape[1]
    return pl.pallas_call(
        _mm_body,
        grid=(n // BLK,),
        in_specs=[pl.BlockSpec((BLK, d), lambda i: (i, 0)),
                  pl.BlockSpec((d, h), lambda i: (0, 0))],
        out_specs=[pl.BlockSpec((BLK, h), lambda i: (i, 0)),
                   pl.BlockSpec((1, 1), lambda i: (0, 0))],
        out_shape=[jax.ShapeDtypeStruct((n, h), jnp.float32),
                   jax.ShapeDtypeStruct((1, 1), jnp.float32)],
    )(x, w)


QSCALE = 1023.0


def _quant_body(y_ref, m_ref, q_ref):
    scale = QSCALE / jnp.maximum(m_ref[0, 0], 1e-30)
    q_ref[...] = jnp.round(y_ref[...] * scale).astype(jnp.int16)


def _quant(y, maxv):
    return pl.pallas_call(
        _quant_body,
        grid=(NBLK,),
        in_specs=[pl.BlockSpec((BLK, H), lambda i: (i, 0)),
                  pl.BlockSpec((1, 1), lambda i: (0, 0))],
        out_specs=pl.BlockSpec((BLK, H), lambda i: (i, 0)),
        out_shape=jax.ShapeDtypeStruct((NPAD, H), jnp.int16),
    )(y, maxv)


def _layer_body(y_ref, a0_ref, a1_ref, b_ref, c1_ref, s1_ref, d1_ref,
                w2_ref, d2_ref, w1n_ref, ynext_ref, pooled_ref, cnt_ref,
                m_ref):
    i = pl.program_id(0)
    y = y_ref[...]
    agg = a0_ref[...].astype(jnp.float32) + a1_ref[...].astype(jnp.float32)
    h1 = jnp.maximum(y * c1_ref[...] + agg * s1_ref[...] + d1_ref[...], 0.0)
    hh = jnp.maximum(
        lax.dot(h1, w2_ref[...],
                preferred_element_type=jnp.float32) + d2_ref[...], 0.0)
    ynext = lax.dot(hh, w1n_ref[...],
                    preferred_element_type=jnp.float32)
    ynext_ref[...] = ynext
    oh = jnp.equal(b_ref[...],
                   lax.broadcasted_iota(jnp.int32, (BLK, G), 1)
                   ).astype(jnp.float32)
    ph = lax.dot_general(oh, hh, (((0,), (0,)), ((), ())),
                         preferred_element_type=jnp.float32)
    pc = lax.dot_general(oh, jnp.ones((BLK, 1), jnp.float32),
                         (((0,), (0,)), ((), ())),
                         preferred_element_type=jnp.float32)

    @pl.when(i == 0)
    def _():
        pooled_ref[...] = jnp.zeros_like(pooled_ref)
        cnt_ref[...] = jnp.zeros_like(cnt_ref)
        m_ref[...] = jnp.zeros_like(m_ref)

    pooled_ref[...] += ph
    cnt_ref[...] += pc
    m_ref[...] = jnp.maximum(m_ref[...], jnp.max(jnp.abs(ynext)))


def _layer_call(y, aggs, batch2d, c1, s1, d1, w2p, d2, w1n):
    return pl.pallas_call(
        _layer_body,
        grid=(NBLK,),
        in_specs=[
            pl.BlockSpec((BLK, H), lambda i: (i, 0)),
            pl.BlockSpec((BLK, H), lambda i: (i, 0)),
            pl.BlockSpec((BLK, H), lambda i: (i + NBLK, 0)),
            pl.BlockSpec((BLK, 1), lambda i: (i, 0)),
            pl.BlockSpec((1, H), lambda i: (0, 0)),
            pl.BlockSpec((1, H), lambda i: (0, 0)),
            pl.BlockSpec((1, H), lambda i: (0, 0)),
            pl.BlockSpec((H, H), lambda i: (0, 0)),
            pl.BlockSpec((1, H), lambda i: (0, 0)),
            pl.BlockSpec((H, H), lambda i: (0, 0)),
        ],
        out_specs=[
            pl.BlockSpec((BLK, H), lambda i: (i, 0)),
            pl.BlockSpec((G, H), lambda i: (0, 0)),
            pl.BlockSpec((G, 1), lambda i: (0, 0)),
            pl.BlockSpec((1, 1), lambda i: (0, 0)),
        ],
        out_shape=[
            jax.ShapeDtypeStruct((NPAD, H), jnp.float32),
            jax.ShapeDtypeStruct((G, H), jnp.float32),
            jax.ShapeDtypeStruct((G, 1), jnp.float32),
            jax.ShapeDtypeStruct((1, 1), jnp.float32),
        ],
    )(y, aggs, aggs, batch2d, c1, s1, d1, w2p, d2, w1n)


def _clf_body(p0_ref, p1_ref, p2_ref, cnt_ref, g_ref,
              wa_ref, wb_ref, wc_ref, wg_ref, b1_ref, w2_ref, b2_ref, o_ref):
    inv = 1.0 / jnp.maximum(cnt_ref[...], 1.0)
    dot = functools.partial(lax.dot,
                            preferred_element_type=jnp.float32)
    e = (dot(p0_ref[...] * inv, wa_ref[...])
         + dot(p1_ref[...] * inv, wb_ref[...])
         + dot(p2_ref[...] * inv, wc_ref[...])
         + dot(g_ref[...], wg_ref[...])
         + b1_ref[...])
    hc = jnp.maximum(e, 0.0)
    o_ref[...] = dot(hc, w2_ref[...]) + b2_ref[...]


def _clf_call(p0, p1, p2, cnt, gp, wa, wb, wc, wg, b1, w2, b2):
    return pl.pallas_call(
        _clf_body,
        out_shape=jax.ShapeDtypeStruct((G, 128), jnp.float32),
    )(p0, p1, p2, cnt, gp, wa, wb, wc, wg, b1, w2, b2)


def kernel(x, edge_index, batch, graph_attr, params):
    layers = params['layers']
    clf = params['clf']

    xp = jnp.zeros((NPAD, D_IN), jnp.float32).at[:N_NODES].set(x)
    src = edge_index[0]
    dst = edge_index[1]
    pad_e = EPAD - E_EDGES
    srcp = jnp.concatenate(
        [src, jnp.zeros((pad_e,), jnp.int32)]).reshape(EPAD // CH, CH)
    dstp = jnp.concatenate(
        [dst, jnp.full((pad_e,), N_NODES, jnp.int32)]).reshape(EPAD // CH, CH)
    batchp = jnp.concatenate(
        [batch, jnp.full((NPAD - N_NODES,), G, jnp.int32)]).reshape(NPAD, 1)
    zeros_blk = jnp.zeros((ROWS_PER_TILE, H), jnp.int16)

    inv_bn = 1.0 / jnp.sqrt(1.0 + BN_EPS)
    pooled = []
    cnt = None
    y, ymax = _matmul(xp, layers[0]['W1'])
    for l in range(3):
        lp = layers[l]
        s1 = (lp['g1'] * inv_bn).reshape(1, H)
        c1 = (1.0 + lp['eps']) * s1
        s1q = s1 * (ymax[0, 0] / QSCALE)
        d1 = (lp['b1'] * lp['g1'] * inv_bn + lp['be1']).reshape(1, H)
        s2 = lp['go'] * inv_bn
        w2p = lp['W2'] * s2[None, :]
        d2 = (lp['b2'] * s2 + lp['bo']).reshape(1, H)
        w1n = layers[l + 1]['W1'] if l < 2 else jnp.zeros((H, H), jnp.float32)
        y16 = _quant(y, ymax)
        aggs = _edge_agg(y16, srcp, dstp, zeros_blk)
        y, psum, pc, ymax = _layer_call(y, aggs, batchp, c1, s1q, d1,
                                        w2p, d2, w1n)
        pooled.append(psum)
        if cnt is None:
            cnt = pc

    wc1 = clf['Wc1']
    wa, wb, wc = wc1[0:H], wc1[H:2 * H], wc1[2 * H:3 * H]
    wg = jnp.zeros((128, H), jnp.float32).at[:GA].set(wc1[3 * H:])
    gp = jnp.zeros((G, 128), jnp.float32).at[:, :GA].set(graph_attr)
    w2 = jnp.zeros((H, 128), jnp.float32).at[:, :2].set(clf['Wc2'])
    b2 = jnp.zeros((1, 128), jnp.float32).at[0, :2].set(clf['bc2'])
    out = _clf_call(pooled[0], pooled[1], pooled[2], cnt, gp,
                    wa, wb, wc, wg, clf['bc1'].reshape(1, H), w2, b2)
    return out[:, :2]


# speedup vs baseline: 1.1496x; 1.0436x over previous
import functools

import jax
import jax.numpy as jnp
from jax import lax
from jax.experimental import pallas as pl
from jax.experimental.pallas import tpu as pltpu
from jax.experimental.pallas import tpu_sc as plsc

N_NODES = 10000
NPAD = 10240
E_EDGES = 320000
D_IN = 128
H = 64
G = 64
GA = 4
BN_EPS = 1e-5

NUM_CORES = 2
NUM_SUBCORES = 16
NUM_TILES = NUM_CORES * NUM_SUBCORES
CH = 128
PER_TILE_CHUNKS = 80
PER_TILE_E = CH * PER_TILE_CHUNKS
EPAD = PER_TILE_E * NUM_TILES
ROWS_PER_TILE = NPAD // NUM_SUBCORES
RING = 8
GAH = 4
SDL = RING - GAH

BLK = 5120
NBLK = NPAD // BLK


@functools.cache
def _edge_agg_kernel():
    mesh = plsc.VectorSubcoreMesh(core_axis_name="c", subcore_axis_name="s")

    @functools.partial(
        pl.kernel,
        out_type=jax.ShapeDtypeStruct((NUM_CORES * NPAD, H), jnp.int16),
        mesh=mesh,
        scratch_types=[
            pltpu.VMEM((PER_TILE_CHUNKS, CH), jnp.int32),
            pltpu.VMEM((PER_TILE_CHUNKS, CH), jnp.int32),
            pltpu.VMEM((RING, CH, H), jnp.int16),
            pltpu.VMEM_SHARED((NPAD, H), jnp.int16),
            pltpu.VMEM_SHARED((NPAD, H), jnp.int16),
            pltpu.SemaphoreType.DMA((RING,)),
            pltpu.SemaphoreType.DMA((RING,)),
        ],
        compiler_params=pltpu.CompilerParams(use_tc_tiling_on_sc=False),
    )
    def body(y_hbm, src_hbm, dst_hbm, zeros_hbm, out_hbm,
             src_all, dst_all, rows, y_sh, acc_sh, gsem, ssem):
        cid = lax.axis_index("c")
        sid = lax.axis_index("s")
        tid = cid * NUM_SUBCORES + sid
        row0 = sid * ROWS_PER_TILE
        ch0 = tid * PER_TILE_CHUNKS
        pltpu.sync_copy(src_hbm.at[pl.ds(ch0, PER_TILE_CHUNKS)], src_all)
        pltpu.sync_copy(dst_hbm.at[pl.ds(ch0, PER_TILE_CHUNKS)], dst_all)
        pltpu.sync_copy(y_hbm.at[pl.ds(row0, ROWS_PER_TILE)],
                        y_sh.at[pl.ds(row0, ROWS_PER_TILE)])
        pltpu.sync_copy(zeros_hbm, acc_sh.at[pl.ds(row0, ROWS_PER_TILE)])

        def gfire(c, b):
            pltpu.async_copy(y_sh.at[src_all.at[c]], rows.at[b], gsem.at[b])

        def gdrain(c, b):
            pltpu.make_async_copy(y_sh.at[src_all.at[c]],
                                  rows.at[b], gsem.at[b]).wait()

        def sfire(c, b):
            pltpu.async_copy(rows.at[b], acc_sh.at[dst_all.at[c]],
                             ssem.at[b], add=True)

        def sdrain(c, b):
            pltpu.make_async_copy(rows.at[b],
                                  acc_sh.at[dst_all.at[c]], ssem.at[b]).wait()

        plsc.subcore_barrier()
        for j in range(GAH):
            gfire(j, j)

        LAST_T = PER_TILE_CHUNKS // RING - 1

        def step(t, carry):
            c0 = t * RING
            for j in range(RING):
                c = c0 + j
                bg = (j + GAH) % RING
                if j >= SDL:
                    sdrain(c - SDL, bg)
                else:
                    @pl.when(t >= 1)
                    def _():
                        sdrain(c - SDL, bg)
                if LAST_T * RING + j + GAH < PER_TILE_CHUNKS:
                    gfire(c + GAH, bg)
                else:
                    @pl.when(t < LAST_T)
                    def _():
                        gfire(c + GAH, bg)
                gdrain(c, j)
                sfire(c, j)
            return carry

        lax.fori_loop(0, LAST_T + 1, step, 0)
        for j in range(SDL):
            c = PER_TILE_CHUNKS - SDL + j
            sdrain(c, c % RING)
        plsc.subcore_barrier()
        out0 = pl.multiple_of(cid * NPAD + row0, 8)
        pltpu.sync_copy(acc_sh.at[pl.ds(row0, ROWS_PER_TILE)],
                        out_hbm.at[pl.ds(out0, ROWS_PER_TILE)])

    return body


def _edge_agg(y, srcp, dstp, zeros_blk):
    return _edge_agg_kernel()(y, srcp, dstp, zeros_blk)


def _mm_body(x_ref, w_ref, o_ref, m_ref):
    i = pl.program_id(0)
    y = lax.dot(x_ref[...], w_ref[...],
                preferred_element_type=jnp.float32)
    o_ref[...] = y

    @pl.when(i == 0)
    def _():
        m_ref[...] = jnp.zeros_like(m_ref)

    m_ref[...] = jnp.maximum(m_ref[...], jnp.max(jnp.abs(y)))


def _matmul(x, w):
    n, d = x.shape
    h = w.shape[1]
    return pl.pallas_call(
        _mm_body,
        grid=(n // BLK,),
        in_specs=[pl.BlockSpec((BLK, d), lambda i: (i, 0)),
                  pl.BlockSpec((d, h), lambda i: (0, 0))],
        out_specs=[pl.BlockSpec((BLK, h), lambda i: (i, 0)),
                   pl.BlockSpec((1, 1), lambda i: (0, 0))],
        out_shape=[jax.ShapeDtypeStruct((n, h), jnp.float32),
                   jax.ShapeDtypeStruct((1, 1), jnp.float32)],
    )(x, w)


QSCALE = 1023.0


def _quant_body(y_ref, m_ref, q_ref):
    scale = QSCALE / jnp.maximum(m_ref[0, 0], 1e-30)
    q_ref[...] = jnp.round(y_ref[...] * scale).astype(jnp.int16)


def _quant(y, maxv):
    return pl.pallas_call(
        _quant_body,
        grid=(NBLK,),
        in_specs=[pl.BlockSpec((BLK, H), lambda i: (i, 0)),
                  pl.BlockSpec((1, 1), lambda i: (0, 0))],
        out_specs=pl.BlockSpec((BLK, H), lambda i: (i, 0)),
        out_shape=jax.ShapeDtypeStruct((NPAD, H), jnp.int16),
    )(y, maxv)


def _layer_body(y_ref, a0_ref, a1_ref, b_ref, c1_ref, s1_ref, d1_ref,
                w2_ref, d2_ref, w1n_ref, ynext_ref, pooled_ref, cnt_ref,
                m_ref):
    i = pl.program_id(0)
    y = y_ref[...]
    agg = a0_ref[...].astype(jnp.float32) + a1_ref[...].astype(jnp.float32)
    h1 = jnp.maximum(y * c1_ref[...] + agg * s1_ref[...] + d1_ref[...], 0.0)
    hh = jnp.maximum(
        lax.dot(h1, w2_ref[...],
                preferred_element_type=jnp.float32) + d2_ref[...], 0.0)
    ynext = lax.dot(hh, w1n_ref[...],
                    preferred_element_type=jnp.float32)
    ynext_ref[...] = ynext
    oh = jnp.equal(b_ref[...],
                   lax.broadcasted_iota(jnp.int32, (BLK, G), 1)
                   ).astype(jnp.float32)
    ph = lax.dot_general(oh, hh, (((0,), (0,)), ((), ())),
                         preferred_element_type=jnp.float32)
    pc = lax.dot_general(oh, jnp.ones((BLK, 1), jnp.float32),
                         (((0,), (0,)), ((), ())),
                         preferred_element_type=jnp.float32)

    @pl.when(i == 0)
    def _():
        pooled_ref[...] = jnp.zeros_like(pooled_ref)
        cnt_ref[...] = jnp.zeros_like(cnt_ref)
        m_ref[...] = jnp.zeros_like(m_ref)

    pooled_ref[...] += ph
    cnt_ref[...] += pc
    m_ref[...] = jnp.maximum(m_ref[...], jnp.max(jnp.abs(ynext)))


def _layer_call(y, aggs, batch2d, c1, s1, d1, w2p, d2, w1n):
    return pl.pallas_call(
        _layer_body,
        grid=(NBLK,),
        in_specs=[
            pl.BlockSpec((BLK, H), lambda i: (i, 0)),
            pl.BlockSpec((BLK, H), lambda i: (i, 0)),
            pl.BlockSpec((BLK, H), lambda i: (i + NBLK, 0)),
            pl.BlockSpec((BLK, 1), lambda i: (i, 0)),
            pl.BlockSpec((1, H), lambda i: (0, 0)),
            pl.BlockSpec((1, H), lambda i: (0, 0)),
            pl.BlockSpec((1, H), lambda i: (0, 0)),
            pl.BlockSpec((H, H), lambda i: (0, 0)),
            pl.BlockSpec((1, H), lambda i: (0, 0)),
            pl.BlockSpec((H, H), lambda i: (0, 0)),
        ],
        out_specs=[
            pl.BlockSpec((BLK, H), lambda i: (i, 0)),
            pl.BlockSpec((G, H), lambda i: (0, 0)),
            pl.BlockSpec((G, 1), lambda i: (0, 0)),
            pl.BlockSpec((1, 1), lambda i: (0, 0)),
        ],
        out_shape=[
            jax.ShapeDtypeStruct((NPAD, H), jnp.float32),
            jax.ShapeDtypeStruct((G, H), jnp.float32),
            jax.ShapeDtypeStruct((G, 1), jnp.float32),
            jax.ShapeDtypeStruct((1, 1), jnp.float32),
        ],
    )(y, aggs, aggs, batch2d, c1, s1, d1, w2p, d2, w1n)


def _clf_body(p0_ref, p1_ref, p2_ref, cnt_ref, g_ref,
              wa_ref, wb_ref, wc_ref, wg_ref, b1_ref, w2_ref, b2_ref, o_ref):
    inv = 1.0 / jnp.maximum(cnt_ref[...], 1.0)
    dot = functools.partial(lax.dot,
                            preferred_element_type=jnp.float32)
    e = (dot(p0_ref[...] * inv, wa_ref[...])
         + dot(p1_ref[...] * inv, wb_ref[...])
         + dot(p2_ref[...] * inv, wc_ref[...])
         + dot(g_ref[...], wg_ref[...])
         + b1_ref[...])
    hc = jnp.maximum(e, 0.0)
    o_ref[...] = dot(hc, w2_ref[...]) + b2_ref[...]


def _clf_call(p0, p1, p2, cnt, gp, wa, wb, wc, wg, b1, w2, b2):
    return pl.pallas_call(
        _clf_body,
        out_shape=jax.ShapeDtypeStruct((G, 128), jnp.float32),
    )(p0, p1, p2, cnt, gp, wa, wb, wc, wg, b1, w2, b2)


def kernel(x, edge_index, batch, graph_attr, params):
    layers = params['layers']
    clf = params['clf']

    xp = jnp.zeros((NPAD, D_IN), jnp.float32).at[:N_NODES].set(x)
    src = edge_index[0]
    dst = edge_index[1]
    pad_e = EPAD - E_EDGES
    srcp = jnp.concatenate(
        [src, jnp.zeros((pad_e,), jnp.int32)]).reshape(EPAD // CH, CH)
    dstp = jnp.concatenate(
        [dst, jnp.full((pad_e,), N_NODES, jnp.int32)]).reshape(EPAD // CH, CH)
    batchp = jnp.concatenate(
        [batch, jnp.full((NPAD - N_NODES,), G, jnp.int32)]).reshape(NPAD, 1)
    zeros_blk = jnp.zeros((ROWS_PER_TILE, H), jnp.int16)

    inv_bn = 1.0 / jnp.sqrt(1.0 + BN_EPS)
    pooled = []
    cnt = None
    y, ymax = _matmul(xp, layers[0]['W1'])
    for l in range(3):
        lp = layers[l]
        s1 = (lp['g1'] * inv_bn).reshape(1, H)
        c1 = (1.0 + lp['eps']) * s1
        s1q = s1 * (ymax[0, 0] / QSCALE)
        d1 = (lp['b1'] * lp['g1'] * inv_bn + lp['be1']).reshape(1, H)
        s2 = lp['go'] * inv_bn
        w2p = lp['W2'] * s2[None, :]
        d2 = (lp['b2'] * s2 + lp['bo']).reshape(1, H)
        w1n = layers[l + 1]['W1'] if l < 2 else jnp.zeros((H, H), jnp.float32)
        y16 = _quant(y, ymax)
        aggs = _edge_agg(y16, srcp, dstp, zeros_blk)
        y, psum, pc, ymax = _layer_call(y, aggs, batchp, c1, s1q, d1,
                                        w2p, d2, w1n)
        pooled.append(psum)
        if cnt is None:
            cnt = pc

    wc1 = clf['Wc1']
    wa, wb, wc = wc1[0:H], wc1[H:2 * H], wc1[2 * H:3 * H]
    wg = jnp.zeros((128, H), jnp.float32).at[:GA].set(wc1[3 * H:])
    gp = jnp.zeros((G, 128), jnp.float32).at[:, :GA].set(graph_attr)
    w2 = jnp.zeros((H, 128), jnp.float32).at[:, :2].set(clf['Wc2'])
    b2 = jnp.zeros((1, 128), jnp.float32).at[0, :2].set(clf['bc2'])
    out = _clf_call(pooled[0], pooled[1], pooled[2], cnt, gp,
                    wa, wb, wc, wg, clf['bc1'].reshape(1, H), w2, b2)
    return out[:, :2]
